# baseline (speedup 1.0000x reference)
import functools

import jax
import jax.numpy as jnp
from jax import lax
from jax.experimental import pallas as pl
from jax.experimental.pallas import tpu as pltpu
from jax.experimental.pallas import tpu_sc as plsc

BS = 16
L = 4096
D = 1024
NCORES = 2
NSUB = 16
CH = 64
DH = D // NCORES
NDB = DH // 16


def _avg_sc(seq, begin, end, inv_cnt):
    mesh = plsc.VectorSubcoreMesh(core_axis_name="c", subcore_axis_name="s")

    @functools.partial(
        pl.kernel,
        mesh=mesh,
        out_type=jax.ShapeDtypeStruct((BS, D), jnp.float32),
        scratch_types=[
            pltpu.VMEM((2 * BS,), jnp.int32),
            pltpu.VMEM((2 * BS,), jnp.int32),
            pltpu.VMEM((2 * BS,), jnp.float32),
            pltpu.VMEM((CH, DH), jnp.float32),
            pltpu.VMEM((CH, DH), jnp.float32),
            pltpu.VMEM((DH,), jnp.float32),
            pltpu.SemaphoreType.DMA,
            pltpu.SemaphoreType.DMA,
        ],
    )
    def k(seq_hbm, begin_hbm, end_hbm, inv_hbm, out_hbm,
          bg_v, en_v, inv_v, buf0, buf1, acc, sem0, sem1):
        c = lax.axis_index("c")
        s = lax.axis_index("s")
        row = s
        d0 = c * DH

        pltpu.sync_copy(begin_hbm, bg_v.at[pl.ds(0, BS)])
        pltpu.sync_copy(end_hbm, en_v.at[pl.ds(0, BS)])
        pltpu.sync_copy(inv_hbm, inv_v.at[pl.ds(0, BS)])
        bg = bg_v[pl.ds(row, 16)][0]
        en = en_v[pl.ds(row, 16)][0]
        inv = inv_v[pl.ds(row, 16)][0]

        cbase0 = (bg // CH) * CH
        nch = (en + CH - 1) // CH - bg // CH

        for db in range(NDB):
            acc[pl.ds(db * 16, 16)] = jnp.zeros((16,), jnp.float32)

        def start_dma(g, buf, sem):
            cb = cbase0 + g * CH
            pltpu.async_copy(
                seq_hbm.at[row, pl.ds(cb, CH), pl.ds(d0, DH)], buf, sem)

        def wait_dma(buf, sem):
            pltpu.make_async_copy(
                seq_hbm.at[row, pl.ds(0, CH), pl.ds(d0, DH)], buf, sem).wait()

        def accumulate(g, buf):
            cb = cbase0 + g * CH
            lo = jnp.maximum(bg - cb, 0)
            hi = jnp.minimum(en - cb, CH)

            def lbody(l, carry):
                for db in range(NDB):
                    sl = pl.ds(db * 16, 16)
                    plsc.addupdate(acc.at[sl], buf[l, sl])
                return carry

            lax.fori_loop(lo, hi, lbody, 0)

        start_dma(0, buf0, sem0)

        @pl.when(nch > 1)
        def _():
            start_dma(1, buf1, sem1)

        def grp_body(grp, carry):
            for j, (buf, sem) in enumerate(((buf0, sem0), (buf1, sem1))):
                g = grp * 2 + j

                @pl.when(g < nch)
                def _():
                    wait_dma(buf, sem)
                    accumulate(g, buf)

                    @pl.when(g + 2 < nch)
                    def _():
                        start_dma(g + 2, buf, sem)
            return carry

        lax.fori_loop(0, (nch + 1) // 2, grp_body, 0)

        for db in range(NDB):
            sl = pl.ds(db * 16, 16)
            acc[sl] = acc[sl] * inv
        pltpu.sync_copy(acc, out_hbm.at[row, pl.ds(d0, DH)])

    return k(seq, begin, end, inv_cnt)


def kernel(seq, begin, end):
    begin = jnp.asarray(begin, jnp.int32)
    end = jnp.asarray(end, jnp.int32)
    inv_cnt = 1.0 / (end - begin).astype(jnp.float32)
    return _avg_sc(seq, begin, end, inv_cnt)


# speedup vs baseline: 3.7012x; 3.7012x over previous
import functools

import jax
import jax.numpy as jnp
from jax import lax
from jax.experimental import pallas as pl
from jax.experimental.pallas import tpu as pltpu
from jax.experimental.pallas import tpu_sc as plsc

BS = 16
L = 4096
D = 1024
NCORES = 2
NSUB = 16
CH = 64
DH = D // NCORES
NDB = DH // 16


def _avg_sc(seq, begin, end, inv_cnt):
    mesh = plsc.VectorSubcoreMesh(core_axis_name="c", subcore_axis_name="s")

    @functools.partial(
        pl.kernel,
        mesh=mesh,
        out_type=jax.ShapeDtypeStruct((BS, D), jnp.float32),
        scratch_types=[
            pltpu.VMEM((2 * BS,), jnp.int32),
            pltpu.VMEM((2 * BS,), jnp.int32),
            pltpu.VMEM((2 * BS,), jnp.float32),
            pltpu.VMEM((CH, DH), jnp.float32),
            pltpu.VMEM((CH, DH), jnp.float32),
            pltpu.VMEM((DH,), jnp.float32),
            pltpu.SemaphoreType.DMA,
            pltpu.SemaphoreType.DMA,
        ],
    )
    def k(seq_hbm, begin_hbm, end_hbm, inv_hbm, out_hbm,
          bg_v, en_v, inv_v, buf0, buf1, acc, sem0, sem1):
        c = lax.axis_index("c")
        s = lax.axis_index("s")
        row = s
        d0 = c * DH

        pltpu.sync_copy(begin_hbm, bg_v.at[pl.ds(0, BS)])
        pltpu.sync_copy(end_hbm, en_v.at[pl.ds(0, BS)])
        pltpu.sync_copy(inv_hbm, inv_v.at[pl.ds(0, BS)])
        bg = bg_v[pl.ds(row, 16)][0]
        en = en_v[pl.ds(row, 16)][0]
        inv = inv_v[pl.ds(row, 16)][0]

        cbase0 = (bg // CH) * CH
        nch = (en + CH - 1) // CH - bg // CH

        def start_dma(g, buf, sem):
            cb = cbase0 + g * CH
            pltpu.async_copy(
                seq_hbm.at[row, pl.ds(cb, CH), pl.ds(d0, DH)], buf, sem)

        def wait_dma(buf, sem):
            pltpu.make_async_copy(
                seq_hbm.at[row, pl.ds(0, CH), pl.ds(d0, DH)], buf, sem).wait()

        def process(g, buf, sem):
            wait_dma(buf, sem)
            cb = cbase0 + g * CH
            lo = jnp.maximum(bg - cb, 0)
            hi = jnp.minimum(en - cb, CH)

            accs = tuple(acc[pl.ds(db * 16, 16)] for db in range(NDB))

            def lbody(l, accs):
                return tuple(
                    a + buf[l, pl.ds(db * 16, 16)]
                    for db, a in enumerate(accs))

            accs = lax.fori_loop(lo, hi, lbody, accs)
            for db, a in enumerate(accs):
                acc[pl.ds(db * 16, 16)] = a

            @pl.when(g + 2 < nch)
            def _():
                start_dma(g + 2, buf, sem)

        for db in range(NDB):
            acc[pl.ds(db * 16, 16)] = jnp.zeros((16,), jnp.float32)

        start_dma(0, buf0, sem0)

        @pl.when(nch > 1)
        def _():
            start_dma(1, buf1, sem1)

        def g_body(g, carry):
            @pl.when(g % 2 == 0)
            def _():
                process(g, buf0, sem0)

            @pl.when(g % 2 == 1)
            def _():
                process(g, buf1, sem1)

            return carry

        lax.fori_loop(0, nch, g_body, 0)

        for db in range(NDB):
            sl = pl.ds(db * 16, 16)
            acc[sl] = acc[sl] * inv
        pltpu.sync_copy(acc, out_hbm.at[row, pl.ds(d0, DH)])

    return k(seq, begin, end, inv_cnt)


def kernel(seq, begin, end):
    begin = jnp.asarray(begin, jnp.int32)
    end = jnp.asarray(end, jnp.int32)
    inv_cnt = 1.0 / (end - begin).astype(jnp.float32)
    return _avg_sc(seq, begin, end, inv_cnt)


# speedup vs baseline: 5.1955x; 1.4038x over previous
import functools

import jax
import jax.numpy as jnp
from jax import lax
from jax.experimental import pallas as pl
from jax.experimental.pallas import tpu as pltpu
from jax.experimental.pallas import tpu_sc as plsc

BS = 16
L = 4096
D = 1024
NCORES = 2
NSUB = 16
CH = 64
DH = D // NCORES
NDB = DH // 16


def _avg_sc(seq, args):
    mesh = plsc.VectorSubcoreMesh(core_axis_name="c", subcore_axis_name="s")

    @functools.partial(
        pl.kernel,
        mesh=mesh,
        out_type=jax.ShapeDtypeStruct((BS, D), jnp.float32),
        scratch_types=[
            pltpu.VMEM((2 * BS,), jnp.int32),
            pltpu.VMEM((2 * BS,), jnp.int32),
            pltpu.VMEM((2 * BS,), jnp.float32),
            pltpu.VMEM((2 * BS,), jnp.int32),
            pltpu.VMEM((2 * BS,), jnp.int32),
            pltpu.VMEM((2 * BS,), jnp.int32),
            pltpu.VMEM((2 * BS,), jnp.int32),
            pltpu.VMEM((CH, DH), jnp.float32),
            pltpu.VMEM((CH, DH), jnp.float32),
            pltpu.VMEM((BS, DH), jnp.float32),
            pltpu.VMEM((DH,), jnp.float32),
            pltpu.VMEM_SHARED((NSUB, BS, DH), jnp.float32),
            pltpu.SemaphoreType.DMA,
            pltpu.SemaphoreType.DMA,
        ],
    )
    def k(seq_hbm, begin_hbm, end_hbm, inv_hbm, cum_hbm, pw_hbm,
          wlo_hbm, whi_hbm, out_hbm,
          bg_v, en_v, inv_v, cum_v, pw_v, wlo_v, whi_v,
          buf0, buf1, part, tmp, shared, sem0, sem1):
        c = lax.axis_index("c")
        s = lax.axis_index("s")
        d0 = c * DH

        for hbm, v in ((begin_hbm, bg_v), (end_hbm, en_v), (inv_hbm, inv_v),
                       (cum_hbm, cum_v), (pw_hbm, pw_v), (wlo_hbm, wlo_v),
                       (whi_hbm, whi_v)):
            pltpu.sync_copy(hbm, v)

        def ext(ref, i):
            return ref[pl.ds(i, 16)][0]

        g0 = ext(pw_v, s)
        g1 = ext(pw_v, s + 1)

        def zero_part(r, carry):
            for db in range(NDB):
                part[r, pl.ds(db * 16, 16)] = jnp.zeros((16,), jnp.float32)
            return carry

        lax.fori_loop(0, BS, zero_part, 0)

        def start_dma(r, cb, buf, sem):
            pltpu.async_copy(
                seq_hbm.at[r, pl.ds(cb, CH), pl.ds(d0, DH)], buf, sem)

        def wait_dma(buf, sem):
            pltpu.make_async_copy(
                seq_hbm.at[0, pl.ds(0, CH), pl.ds(d0, DH)], buf, sem).wait()

        def chunk(r, cb0, g, nch, lo_abs, hi_abs, buf, sem):
            wait_dma(buf, sem)
            cb = cb0 + g * CH
            lo = jnp.maximum(lo_abs - cb, 0)
            hi = jnp.minimum(hi_abs - cb, CH)

            accs = tuple(part[r, pl.ds(db * 16, 16)] for db in range(NDB))

            def add_l(l, accs):
                return tuple(
                    a + buf[l, pl.ds(db * 16, 16)]
                    for db, a in enumerate(accs))

            n2 = (hi - lo) // 2

            def pair_body(i, accs):
                l = lo + 2 * i
                return add_l(l + 1, add_l(l, accs))

            accs = lax.fori_loop(0, n2, pair_body, accs)
            accs = lax.fori_loop(lo + 2 * n2, hi, add_l, accs)

            for db, a in enumerate(accs):
                part[r, pl.ds(db * 16, 16)] = a

            @pl.when(g + 2 < nch)
            def _():
                start_dma(r, cb0 + (g + 2) * CH, buf, sem)

        def seg_bounds(r):
            S = ext(cum_v, r)
            bg_r = ext(bg_v, r)
            ln = ext(en_v, r) - bg_r
            a = jnp.maximum(g0 - S, 0)
            b = jnp.minimum(g1 - S, ln)
            return bg_r, a, b

        def seg_body(r, carry):
            bg_r, a, b = seg_bounds(r)

            @pl.when(a < b)
            def _():
                lo_abs = bg_r + a
                hi_abs = bg_r + b
                cb0 = (lo_abs // CH) * CH
                nch = (hi_abs + CH - 1) // CH - lo_abs // CH
                start_dma(r, cb0, buf0, sem0)

                @pl.when(nch > 1)
                def _():
                    start_dma(r, cb0 + CH, buf1, sem1)

                def g_body(g, carry2):
                    @pl.when(g % 2 == 0)
                    def _():
                        chunk(r, cb0, g, nch, lo_abs, hi_abs, buf0, sem0)

                    @pl.when(g % 2 == 1)
                    def _():
                        chunk(r, cb0, g, nch, lo_abs, hi_abs, buf1, sem1)

                    return carry2

                lax.fori_loop(0, nch, g_body, 0)

            return carry

        lax.fori_loop(0, BS, seg_body, 0)

        def copy_body(r, carry):
            _, a, b = seg_bounds(r)

            @pl.when(a < b)
            def _():
                pltpu.sync_copy(part.at[r], shared.at[s, r])

            return carry

        lax.fori_loop(0, BS, copy_body, 0)
        plsc.subcore_barrier()

        wlo = ext(wlo_v, s)
        whi = ext(whi_v, s)
        accs = tuple(jnp.zeros((16,), jnp.float32) for _ in range(NDB))

        def fin_body(w, accs):
            pltpu.sync_copy(shared.at[w, s], tmp)
            return tuple(
                a + tmp[pl.ds(db * 16, 16)] for db, a in enumerate(accs))

        accs = lax.fori_loop(wlo, whi + 1, fin_body, accs)
        inv = ext(inv_v, s)
        for db, a in enumerate(accs):
            tmp[pl.ds(db * 16, 16)] = a * inv
        pltpu.sync_copy(tmp, out_hbm.at[s, pl.ds(d0, DH)])

    return k(seq, *args)


def kernel(seq, begin, end):
    begin = jnp.asarray(begin, jnp.int32)
    end = jnp.asarray(end, jnp.int32)
    lens = end - begin
    cum = jnp.concatenate([jnp.zeros((1,), jnp.int32), jnp.cumsum(lens)])
    total = cum[BS]
    pw = (jnp.arange(NSUB + 1, dtype=jnp.int32) * total) // NSUB
    wlo = (NSUB * cum[:BS]) // total
    whi = jnp.minimum(NSUB - 1, (NSUB * cum[1:BS + 1] - 1) // total)
    inv_cnt = 1.0 / lens.astype(jnp.float32)

    def pad32(x):
        return jnp.zeros((2 * BS,), x.dtype).at[: x.shape[0]].set(x)

    args = tuple(pad32(x.astype(jnp.int32)) if x.dtype != jnp.float32
                 else pad32(x)
                 for x in (begin, end, inv_cnt, cum, pw, wlo, whi))
    return _avg_sc(seq, args)
